# Initial kernel scaffold; baseline (speedup 1.0000x reference)
import jax
import jax.numpy as jnp
from jax.experimental import pallas as pl


def kernel(node_features, edge_features, node_down_W, node_down_b, edge_down_W, edge_down_b, q_W, q_b, k_W, k_b, v_W, v_b, o_W, o_b, sn_g, sn_b, node_up_W, node_up_b, edge_up_W, edge_up_b, node_gate, edge_gate, nn_g, nn_b, en_g, en_b, edge_index):
    raise NotImplementedError("write your pallas kernel here")


# speedup vs baseline: 130.1387x; 130.1387x over previous
import functools
import math

import jax
import jax.numpy as jnp
from jax import lax
from jax.experimental import pallas as pl
from jax.experimental.pallas import tpu as pltpu
from jax.experimental.pallas import tpu_sc as plsc

N = 10000
E = 160000
DN = 256
DE = 16
DL = 128
H = 4
DH = 32
QKV = 3 * DL

E_PAD = 163840
BME = 2048
GE = E_PAD // BME
N_PAD = 10240
BMN = 2048
GN = N_PAD // BMN

_NC = 2
_NS = 16
_NW = _NC * _NS
_CH = 128
_PER_W = E_PAD // _NW
_NCH = _PER_W // _CH
_NPS = N_PAD // _NS

_INV = 1.0 / math.sqrt(DH)
_f32 = jnp.float32


def _hsel():
    return (lax.broadcasted_iota(jnp.int32, (DL, H), 0) // DH ==
            lax.broadcasted_iota(jnp.int32, (DL, H), 1)).astype(_f32)


def _hselT():
    return (lax.broadcasted_iota(jnp.int32, (H, DL), 0) ==
            lax.broadcasted_iota(jnp.int32, (H, DL), 1) // DH).astype(_f32)


def _dsel():
    return (lax.broadcasted_iota(jnp.int32, (H, DL), 0) ==
            lax.broadcasted_iota(jnp.int32, (H, DL), 1)).astype(_f32)


def _densel():
    return (lax.broadcasted_iota(jnp.int32, (DL, DL), 0) ==
            lax.broadcasted_iota(jnp.int32, (DL, DL), 1) // DH).astype(_f32)


def _proj_body(x_ref, w1_ref, b1_ref, wq_ref, bq_ref, lat_ref, qkv_ref):
    lat = jnp.dot(x_ref[...], w1_ref[...],
                  preferred_element_type=_f32) + b1_ref[0:1, :]
    lat_ref[...] = lat
    qkv_ref[...] = jnp.dot(lat, wq_ref[...],
                           preferred_element_type=_f32) + bq_ref[0:1, :]


def _attn_body(qkv_ref, gs_ref, gt_ref, oe_ref, sst_ref, bm_ref):
    qkv = qkv_ref[...]
    qe = qkv[:, :DL]
    ke = qkv[:, DL:2 * DL]
    gs = gs_ref[...]
    gt = gt_ref[...]
    hsel = _hsel()
    hselT = _hselT()

    def hd(a, b):
        return jnp.dot(a * b, hsel, preferred_element_type=_f32) * _INV

    s1 = hd(qe, gs[:, DL:2 * DL])
    s2 = hd(qe, gt[:, DL:2 * DL])
    m = jnp.maximum(jnp.maximum(s1, s2), -1e9)
    a1 = jnp.exp(s1 - m)
    a2 = jnp.exp(s2 - m)
    r = 1.0 / (a1 + a2 + 1e-10)
    w1 = jnp.dot(a1 * r, hselT, preferred_element_type=_f32)
    w2 = jnp.dot(a2 * r, hselT, preferred_element_type=_f32)
    oe_ref[...] = w1 * gs[:, 2 * DL:] + w2 * gt[:, 2 * DL:]
    ss = hd(gs[:, :DL], ke)
    st = hd(gt[:, :DL], ke)
    sst = jnp.concatenate([ss, st], axis=1)
    sst_ref[...] = sst
    bm_ref[...] = jnp.max(sst, axis=0, keepdims=True).reshape(1, 1, 2 * H)


def _expand_body(sst_ref, v_ref, m_ref, out_ref):
    i = pl.program_id(0)
    sst = sst_ref[...]
    p = jnp.exp(sst - m_ref[0:1, :])
    rows = i * BME + lax.broadcasted_iota(jnp.int32, (BME, 2 * H), 0)
    p = jnp.where(rows < E, p, 0.0)
    ps = p[:, :H]
    pt = p[:, H:]
    hselT = _hselT()
    dsel = _dsel()
    v = v_ref[...]
    out_ref[0] = jnp.dot(ps, hselT, preferred_element_type=_f32) * v
    out_ref[1] = jnp.dot(pt, hselT, preferred_element_type=_f32) * v
    out_ref[2] = jnp.dot(ps, dsel, preferred_element_type=_f32)
    out_ref[3] = jnp.dot(pt, dsel, preferred_element_type=_f32)


def _ln(y, g, b):
    mu = jnp.mean(y, axis=1, keepdims=True)
    var = jnp.mean((y - mu) ** 2, axis=1, keepdims=True)
    return (y - mu) / jnp.sqrt(var + 1e-5) * g + b


def _node_fin_body(num_ref, den_ref, lat_ref, nf_ref, owt_ref, ob_ref,
                   sng_ref, snb_ref, upw_ref, upb_ref, nng_ref, nnb_ref,
                   out_ref):
    num = num_ref[...]
    den = den_ref[...]
    denf = jnp.dot(den, _densel(), preferred_element_type=_f32)
    on = num / (denf + 1e-20)
    x = jnp.dot(on, owt_ref[...], preferred_element_type=_f32) + ob_ref[0:1, :]
    z = _ln(lat_ref[...] + x, sng_ref[0:1, :], snb_ref[0:1, :])
    nu = jnp.dot(z, upw_ref[...], preferred_element_type=_f32) + upb_ref[0:1, :]
    out_ref[...] = _ln(nf_ref[...] + nu, nng_ref[0:1, :], nnb_ref[0:1, :])


def _edge_fin_body(oe_ref, lat_ref, ef_ref, owt_ref, ob_ref,
                   sng_ref, snb_ref, upw_ref, upb_ref, eng_ref, enb_ref,
                   out_ref):
    x = jnp.dot(oe_ref[...], owt_ref[...],
                preferred_element_type=_f32) + ob_ref[0:1, :]
    z = _ln(lat_ref[...] + x, sng_ref[0:1, :], snb_ref[0:1, :])
    eu = jnp.dot(z, upw_ref[...], preferred_element_type=_f32) + upb_ref[0:1, :]
    out_ref[...] = _ln(ef_ref[...] + eu, eng_ref[0:1, :], enb_ref[0:1, :])


def _sc_gather_body(tab, src, tgt, gs, gt, idx_v, rows_v, sem):
    cid = lax.axis_index("c")
    sid = lax.axis_index("s")
    wid = sid * _NC + cid
    base = wid * _PER_W

    def one(idx_hbm, out_hbm):
        def step(j, c):
            off = base + j * _CH
            pltpu.sync_copy(idx_hbm.at[pl.ds(off, _CH)], idx_v)
            pltpu.async_copy(tab.at[idx_v], rows_v, sem).wait()
            pltpu.sync_copy(rows_v, out_hbm.at[pl.ds(off, _CH)])
            return c
        lax.fori_loop(0, _NCH, step, 0)

    one(src, gs)
    one(tgt, gt)


def _sc_scatter_body(rows4, idx2, z128, out, idx_v, rows_v, tab):
    cid = lax.axis_index("c")
    sid = lax.axis_index("s")
    r0 = sid * _NPS
    def zstep(j, c):
        rr = r0 + j * _CH
        pltpu.sync_copy(z128.at[pl.ds(rr, _CH)], rows_v)
        pltpu.sync_copy(rows_v, tab.at[pl.ds(rr, _CH)])
        return c
    lax.fori_loop(0, _NPS // _CH, zstep, 0)
    plsc.subcore_barrier()
    base = sid * (E_PAD // _NS)

    def one(part):
        ro = (2 * cid + part) * E_PAD + base
        io = part * E_PAD + base
        def step(j, c):
            pltpu.sync_copy(idx2.at[pl.ds(io + j * _CH, _CH)], idx_v)
            pltpu.sync_copy(rows4.at[pl.ds(ro + j * _CH, _CH)], rows_v)
            pltpu.sync_copy(rows_v, tab.at[idx_v], add=True)
            return c
        lax.fori_loop(0, (E_PAD // _NS) // _CH, step, 0)

    one(0)
    one(1)
    plsc.subcore_barrier()
    def wstep(j, c):
        rr = r0 + j * _CH
        pltpu.sync_copy(tab.at[pl.ds(rr, _CH)], rows_v)
        pltpu.sync_copy(rows_v, out.at[pl.ds(cid * N_PAD + rr, _CH)])
        return c
    lax.fori_loop(0, _NPS // _CH, wstep, 0)


@functools.lru_cache(maxsize=None)
def _sc_kernels():
    mesh = plsc.VectorSubcoreMesh(core_axis_name="c", subcore_axis_name="s",
                                  num_cores=_NC, num_subcores=_NS)
    gather = pl.kernel(
        _sc_gather_body,
        out_type=(jax.ShapeDtypeStruct((E_PAD, QKV), _f32),
                  jax.ShapeDtypeStruct((E_PAD, QKV), _f32)),
        mesh=mesh,
        scratch_types=[
            pltpu.VMEM((_CH,), jnp.int32),
            pltpu.VMEM((_CH, QKV), _f32),
            pltpu.SemaphoreType.DMA,
        ],
    )
    scatter = pl.kernel(
        _sc_scatter_body,
        out_type=jax.ShapeDtypeStruct((2 * N_PAD, DL), _f32),
        mesh=mesh,
        scratch_types=[
            pltpu.VMEM((_CH,), jnp.int32),
            pltpu.VMEM((_CH, DL), _f32),
            pltpu.VMEM_SHARED((N_PAD, DL), _f32),
        ],
    )
    return gather, scatter


def kernel(node_features, edge_features, node_down_W, node_down_b,
           edge_down_W, edge_down_b, q_W, q_b, k_W, k_b, v_W, v_b,
           o_W, o_b, sn_g, sn_b, node_up_W, node_up_b, edge_up_W, edge_up_b,
           node_gate, edge_gate, nn_g, nn_b, en_g, en_b, edge_index):
    def bc(v, w):
        return jnp.broadcast_to(v.reshape(1, w), (8, w))

    Wqkvt = jnp.concatenate([q_W.T, k_W.T, v_W.T], axis=1)
    bqkv = bc(jnp.concatenate([q_b, k_b, v_b]), QKV)
    g_n = jax.nn.sigmoid(node_gate)[0]
    g_e = jax.nn.sigmoid(edge_gate)[0]
    nupWt = node_up_W.T * g_n
    nupb = node_up_b * g_n
    eupWt = edge_up_W.T * g_e
    eupb = edge_up_b * g_e
    ef_pad = jnp.concatenate(
        [edge_features, jnp.zeros((E_PAD - E, DE), _f32)], axis=0)
    nf_pad = jnp.concatenate(
        [node_features, jnp.zeros((N_PAD - N, DN), _f32)], axis=0)
    src = edge_index[0]
    tgt = edge_index[1]
    zpad = jnp.zeros((E_PAD - E,), jnp.int32)
    src_p = jnp.concatenate([src, zpad])
    tgt_p = jnp.concatenate([tgt, zpad])

    full = lambda shape: pl.BlockSpec(shape, lambda i: (0, 0))

    nl, qkvn = pl.pallas_call(
        _proj_body,
        grid=(GN,),
        in_specs=[pl.BlockSpec((BMN, DN), lambda i: (i, 0)),
                  full((DN, DL)), full((8, DL)),
                  full((DL, QKV)), full((8, QKV))],
        out_specs=[pl.BlockSpec((BMN, DL), lambda i: (i, 0)),
                   pl.BlockSpec((BMN, QKV), lambda i: (i, 0))],
        out_shape=[jax.ShapeDtypeStruct((N_PAD, DL), _f32),
                   jax.ShapeDtypeStruct((N_PAD, QKV), _f32)],
    )(nf_pad, node_down_W.T, bc(node_down_b, DL), Wqkvt, bqkv)

    el, qkve = pl.pallas_call(
        _proj_body,
        grid=(GE,),
        in_specs=[pl.BlockSpec((BME, DE), lambda i: (i, 0)),
                  full((DE, DL)), full((8, DL)),
                  full((DL, QKV)), full((8, QKV))],
        out_specs=[pl.BlockSpec((BME, DL), lambda i: (i, 0)),
                   pl.BlockSpec((BME, QKV), lambda i: (i, 0))],
        out_shape=[jax.ShapeDtypeStruct((E_PAD, DL), _f32),
                   jax.ShapeDtypeStruct((E_PAD, QKV), _f32)],
    )(ef_pad, edge_down_W.T, bc(edge_down_b, DL), Wqkvt, bqkv)

    sc_gather, sc_scatter = _sc_kernels()
    gs, gt = sc_gather(qkvn, src_p, tgt_p)

    oe, sst, bmax = pl.pallas_call(
        _attn_body,
        grid=(GE,),
        in_specs=[pl.BlockSpec((BME, QKV), lambda i: (i, 0)),
                  pl.BlockSpec((BME, QKV), lambda i: (i, 0)),
                  pl.BlockSpec((BME, QKV), lambda i: (i, 0))],
        out_specs=[pl.BlockSpec((BME, DL), lambda i: (i, 0)),
                   pl.BlockSpec((BME, 2 * H), lambda i: (i, 0)),
                   pl.BlockSpec((1, 1, 2 * H), lambda i: (i, 0, 0))],
        out_shape=[jax.ShapeDtypeStruct((E_PAD, DL), _f32),
                   jax.ShapeDtypeStruct((E_PAD, 2 * H), _f32),
                   jax.ShapeDtypeStruct((GE, 1, 2 * H), _f32)],
    )(qkve, gs, gt)

    mm = jnp.max(bmax, axis=(0, 1))
    m4 = jnp.maximum(mm[:H], mm[H:])
    m8 = bc(jnp.concatenate([m4, m4]), 2 * H)

    rows4 = pl.pallas_call(
        _expand_body,
        grid=(GE,),
        in_specs=[pl.BlockSpec((BME, 2 * H), lambda i: (i, 0)),
                  pl.BlockSpec((BME, DL), lambda i: (i, 2)),
                  full((8, 2 * H))],
        out_specs=pl.BlockSpec((4, BME, DL), lambda i: (0, i, 0)),
        out_shape=jax.ShapeDtypeStruct((4, E_PAD, DL), _f32),
    )(sst, qkve, m8)

    idx2 = jnp.concatenate([src_p, tgt_p])
    numden = sc_scatter(rows4.reshape(4 * E_PAD, DL), idx2,
                        jnp.zeros((N_PAD, DL), _f32))

    nn_pad = pl.pallas_call(
        _node_fin_body,
        grid=(GN,),
        in_specs=[pl.BlockSpec((BMN, DL), lambda i: (i, 0)),
                  pl.BlockSpec((BMN, DL), lambda i: (i + N_PAD // BMN, 0)),
                  pl.BlockSpec((BMN, DL), lambda i: (i, 0)),
                  pl.BlockSpec((BMN, DN), lambda i: (i, 0)),
                  full((DL, DL)), full((8, DL)),
                  full((8, DL)), full((8, DL)),
                  full((DL, DN)), full((8, DN)),
                  full((8, DN)), full((8, DN))],
        out_specs=pl.BlockSpec((BMN, DN), lambda i: (i, 0)),
        out_shape=jax.ShapeDtypeStruct((N_PAD, DN), _f32),
    )(numden, numden, nl, nf_pad, o_W.T, bc(o_b, DL),
      bc(sn_g, DL), bc(sn_b, DL), nupWt, bc(nupb, DN),
      bc(nn_g, DN), bc(nn_b, DN))

    ne_pad = pl.pallas_call(
        _edge_fin_body,
        grid=(GE,),
        in_specs=[pl.BlockSpec((BME, DL), lambda i: (i, 0)),
                  pl.BlockSpec((BME, DL), lambda i: (i, 0)),
                  pl.BlockSpec((BME, DE), lambda i: (i, 0)),
                  full((DL, DL)), full((8, DL)),
                  full((8, DL)), full((8, DL)),
                  full((DL, DE)), full((8, DE)),
                  full((8, DE)), full((8, DE))],
        out_specs=pl.BlockSpec((BME, DE), lambda i: (i, 0)),
        out_shape=jax.ShapeDtypeStruct((E_PAD, DE), _f32),
    )(oe, el, ef_pad, o_W.T, bc(o_b, DL),
      bc(sn_g, DL), bc(sn_b, DL), eupWt, bc(eupb, DE),
      bc(en_g, DE), bc(en_b, DE))

    return (nn_pad[:N], ne_pad[:E])


# speedup vs baseline: 148.1659x; 1.1385x over previous
import functools
import math

import jax
import jax.numpy as jnp
from jax import lax
from jax.experimental import pallas as pl
from jax.experimental.pallas import tpu as pltpu
from jax.experimental.pallas import tpu_sc as plsc

N = 10000
E = 160000
DN = 256
DE = 16
DL = 128
H = 4
DH = 32
QKV = 3 * DL

E_PAD = 163840
BME = 2048
GE = E_PAD // BME
N_PAD = 10240
BMN = 2048
GN = N_PAD // BMN

_NC = 2
_NS = 16
_NW = _NC * _NS
_CH = 128
_PER_W = E_PAD // _NW
_NCH = _PER_W // _CH
_NPS = N_PAD // _NS

_INV = 1.0 / math.sqrt(DH)
_f32 = jnp.float32


def _hsel():
    return (lax.broadcasted_iota(jnp.int32, (DL, H), 0) // DH ==
            lax.broadcasted_iota(jnp.int32, (DL, H), 1)).astype(_f32)


def _hselT():
    return (lax.broadcasted_iota(jnp.int32, (H, DL), 0) ==
            lax.broadcasted_iota(jnp.int32, (H, DL), 1) // DH).astype(_f32)


def _dsel():
    return (lax.broadcasted_iota(jnp.int32, (H, DL), 0) ==
            lax.broadcasted_iota(jnp.int32, (H, DL), 1)).astype(_f32)


def _densel():
    return (lax.broadcasted_iota(jnp.int32, (DL, DL), 0) ==
            lax.broadcasted_iota(jnp.int32, (DL, DL), 1) // DH).astype(_f32)


def _proj_body(x_ref, w1_ref, b1_ref, wq_ref, bq_ref, lat_ref, qkv_ref):
    lat = jnp.dot(x_ref[...], w1_ref[...],
                  preferred_element_type=_f32) + b1_ref[0:1, :]
    lat_ref[...] = lat
    qkv_ref[...] = jnp.dot(lat, wq_ref[...],
                           preferred_element_type=_f32) + bq_ref[0:1, :]


def _attn_body(qkv_ref, gs_ref, gt_ref, oe_ref, sst_ref, bm_ref):
    qkv = qkv_ref[...]
    qe = qkv[:, :DL]
    ke = qkv[:, DL:2 * DL]
    gs = gs_ref[...]
    gt = gt_ref[...]
    hsel = _hsel()
    hselT = _hselT()

    def hd(a, b):
        return jnp.dot(a * b, hsel, preferred_element_type=_f32) * _INV

    s1 = hd(qe, gs[:, DL:2 * DL])
    s2 = hd(qe, gt[:, DL:2 * DL])
    m = jnp.maximum(jnp.maximum(s1, s2), -1e9)
    a1 = jnp.exp(s1 - m)
    a2 = jnp.exp(s2 - m)
    r = 1.0 / (a1 + a2 + 1e-10)
    w1 = jnp.dot(a1 * r, hselT, preferred_element_type=_f32)
    w2 = jnp.dot(a2 * r, hselT, preferred_element_type=_f32)
    oe_ref[...] = w1 * gs[:, 2 * DL:] + w2 * gt[:, 2 * DL:]
    ss = hd(gs[:, :DL], ke)
    st = hd(gt[:, :DL], ke)
    sst = jnp.concatenate([ss, st], axis=1)
    sst_ref[...] = sst
    bm_ref[...] = jnp.max(sst, axis=0, keepdims=True).reshape(1, 1, 2 * H)


def _expand_body(sst_ref, v_ref, m_ref, out_ref):
    i = pl.program_id(0)
    sst = sst_ref[...]
    p = jnp.exp(sst - m_ref[0:1, :])
    rows = i * BME + lax.broadcasted_iota(jnp.int32, (BME, 2 * H), 0)
    p = jnp.where(rows < E, p, 0.0)
    ps = p[:, :H]
    pt = p[:, H:]
    hselT = _hselT()
    dsel = _dsel()
    v = v_ref[...]
    out_ref[0] = jnp.dot(ps, hselT, preferred_element_type=_f32) * v
    out_ref[1] = jnp.dot(pt, hselT, preferred_element_type=_f32) * v
    out_ref[2] = jnp.dot(ps, dsel, preferred_element_type=_f32)
    out_ref[3] = jnp.dot(pt, dsel, preferred_element_type=_f32)


def _ln(y, g, b):
    mu = jnp.mean(y, axis=1, keepdims=True)
    var = jnp.mean((y - mu) ** 2, axis=1, keepdims=True)
    return (y - mu) / jnp.sqrt(var + 1e-5) * g + b


def _node_fin_body(num_ref, den_ref, lat_ref, nf_ref, owt_ref, ob_ref,
                   sng_ref, snb_ref, upw_ref, upb_ref, nng_ref, nnb_ref,
                   out_ref):
    num = num_ref[...]
    den = den_ref[...]
    denf = jnp.dot(den, _densel(), preferred_element_type=_f32)
    on = num / (denf + 1e-20)
    x = jnp.dot(on, owt_ref[...], preferred_element_type=_f32) + ob_ref[0:1, :]
    z = _ln(lat_ref[...] + x, sng_ref[0:1, :], snb_ref[0:1, :])
    nu = jnp.dot(z, upw_ref[...], preferred_element_type=_f32) + upb_ref[0:1, :]
    out_ref[...] = _ln(nf_ref[...] + nu, nng_ref[0:1, :], nnb_ref[0:1, :])


def _edge_fin_body(oe_ref, lat_ref, ef_ref, owt_ref, ob_ref,
                   sng_ref, snb_ref, upw_ref, upb_ref, eng_ref, enb_ref,
                   out_ref):
    x = jnp.dot(oe_ref[...], owt_ref[...],
                preferred_element_type=_f32) + ob_ref[0:1, :]
    z = _ln(lat_ref[...] + x, sng_ref[0:1, :], snb_ref[0:1, :])
    eu = jnp.dot(z, upw_ref[...], preferred_element_type=_f32) + upb_ref[0:1, :]
    out_ref[...] = _ln(ef_ref[...] + eu, eng_ref[0:1, :], enb_ref[0:1, :])


def _sc_gather_body(tab, idx2, g2, i0, i1, r0b, r1b, gs0, gs1, ws0, ws1):
    cid = lax.axis_index("c")
    sid = lax.axis_index("s")
    wid = sid * _NC + cid
    base = wid * (2 * _PER_W)
    nch = (2 * _PER_W) // _CH
    ibufs = (i0, i1)
    rbufs = (r0b, r1b)
    gsems = (gs0, gs1)
    wsems = (ws0, ws1)

    pltpu.sync_copy(idx2.at[pl.ds(base, _CH)], i0)
    pltpu.async_copy(tab.at[i0], r0b, gs0)

    def outer(jj, c):
        for b in range(2):
            j = jj * 2 + b
            nb = 1 - b
            @pl.when(j + 1 < nch)
            def _():
                pltpu.sync_copy(idx2.at[pl.ds(base + (j + 1) * _CH, _CH)],
                                ibufs[nb])
            @pl.when(j >= 1)
            def _():
                pltpu.make_async_copy(rbufs[nb],
                                      g2.at[pl.ds(base, _CH)],
                                      wsems[nb]).wait()
            @pl.when(j + 1 < nch)
            def _():
                pltpu.async_copy(tab.at[ibufs[nb]], rbufs[nb], gsems[nb])
            pltpu.make_async_copy(tab.at[ibufs[b]], rbufs[b],
                                  gsems[b]).wait()
            pltpu.async_copy(rbufs[b], g2.at[pl.ds(base + j * _CH, _CH)],
                             wsems[b])
        return c

    lax.fori_loop(0, nch // 2, outer, 0)
    pltpu.make_async_copy(r1b, g2.at[pl.ds(base, _CH)], ws1).wait()


def _sc_scatter_body(rows4, idx2, z128, out, idx_v, rows_v, tab):
    cid = lax.axis_index("c")
    sid = lax.axis_index("s")
    r0 = sid * _NPS
    def zstep(j, c):
        rr = r0 + j * _CH
        pltpu.sync_copy(z128.at[pl.ds(rr, _CH)], rows_v)
        pltpu.sync_copy(rows_v, tab.at[pl.ds(rr, _CH)])
        return c
    lax.fori_loop(0, _NPS // _CH, zstep, 0)
    plsc.subcore_barrier()
    base = sid * (E_PAD // _NS)

    def one(part):
        ro = (2 * cid + part) * E_PAD + base
        io = part * E_PAD + base
        def step(j, c):
            pltpu.sync_copy(idx2.at[pl.ds(io + j * _CH, _CH)], idx_v)
            pltpu.sync_copy(rows4.at[pl.ds(ro + j * _CH, _CH)], rows_v)
            pltpu.sync_copy(rows_v, tab.at[idx_v], add=True)
            return c
        lax.fori_loop(0, (E_PAD // _NS) // _CH, step, 0)

    one(0)
    one(1)
    plsc.subcore_barrier()
    def wstep(j, c):
        rr = r0 + j * _CH
        pltpu.sync_copy(tab.at[pl.ds(rr, _CH)], rows_v)
        pltpu.sync_copy(rows_v, out.at[pl.ds(cid * N_PAD + rr, _CH)])
        return c
    lax.fori_loop(0, _NPS // _CH, wstep, 0)


@functools.lru_cache(maxsize=None)
def _sc_kernels():
    mesh = plsc.VectorSubcoreMesh(core_axis_name="c", subcore_axis_name="s",
                                  num_cores=_NC, num_subcores=_NS)
    gather = pl.kernel(
        _sc_gather_body,
        out_type=jax.ShapeDtypeStruct((2 * E_PAD, QKV), _f32),
        mesh=mesh,
        scratch_types=[
            pltpu.VMEM((_CH,), jnp.int32),
            pltpu.VMEM((_CH,), jnp.int32),
            pltpu.VMEM((_CH, QKV), _f32),
            pltpu.VMEM((_CH, QKV), _f32),
            pltpu.SemaphoreType.DMA,
            pltpu.SemaphoreType.DMA,
            pltpu.SemaphoreType.DMA,
            pltpu.SemaphoreType.DMA,
        ],
    )
    scatter = pl.kernel(
        _sc_scatter_body,
        out_type=jax.ShapeDtypeStruct((2 * N_PAD, DL), _f32),
        mesh=mesh,
        scratch_types=[
            pltpu.VMEM((_CH,), jnp.int32),
            pltpu.VMEM((_CH, DL), _f32),
            pltpu.VMEM_SHARED((N_PAD, DL), _f32),
        ],
    )
    return gather, scatter


def kernel(node_features, edge_features, node_down_W, node_down_b,
           edge_down_W, edge_down_b, q_W, q_b, k_W, k_b, v_W, v_b,
           o_W, o_b, sn_g, sn_b, node_up_W, node_up_b, edge_up_W, edge_up_b,
           node_gate, edge_gate, nn_g, nn_b, en_g, en_b, edge_index):
    def bc(v, w):
        return jnp.broadcast_to(v.reshape(1, w), (8, w))

    Wqkvt = jnp.concatenate([q_W.T, k_W.T, v_W.T], axis=1)
    bqkv = bc(jnp.concatenate([q_b, k_b, v_b]), QKV)
    g_n = jax.nn.sigmoid(node_gate)[0]
    g_e = jax.nn.sigmoid(edge_gate)[0]
    nupWt = node_up_W.T * g_n
    nupb = node_up_b * g_n
    eupWt = edge_up_W.T * g_e
    eupb = edge_up_b * g_e
    ef_pad = jnp.concatenate(
        [edge_features, jnp.zeros((E_PAD - E, DE), _f32)], axis=0)
    nf_pad = jnp.concatenate(
        [node_features, jnp.zeros((N_PAD - N, DN), _f32)], axis=0)
    src = edge_index[0]
    tgt = edge_index[1]
    zpad = jnp.zeros((E_PAD - E,), jnp.int32)
    src_p = jnp.concatenate([src, zpad])
    tgt_p = jnp.concatenate([tgt, zpad])

    full = lambda shape: pl.BlockSpec(shape, lambda i: (0, 0))

    nl, qkvn = pl.pallas_call(
        _proj_body,
        grid=(GN,),
        in_specs=[pl.BlockSpec((BMN, DN), lambda i: (i, 0)),
                  full((DN, DL)), full((8, DL)),
                  full((DL, QKV)), full((8, QKV))],
        out_specs=[pl.BlockSpec((BMN, DL), lambda i: (i, 0)),
                   pl.BlockSpec((BMN, QKV), lambda i: (i, 0))],
        out_shape=[jax.ShapeDtypeStruct((N_PAD, DL), _f32),
                   jax.ShapeDtypeStruct((N_PAD, QKV), _f32)],
    )(nf_pad, node_down_W.T, bc(node_down_b, DL), Wqkvt, bqkv)

    el, qkve = pl.pallas_call(
        _proj_body,
        grid=(GE,),
        in_specs=[pl.BlockSpec((BME, DE), lambda i: (i, 0)),
                  full((DE, DL)), full((8, DL)),
                  full((DL, QKV)), full((8, QKV))],
        out_specs=[pl.BlockSpec((BME, DL), lambda i: (i, 0)),
                   pl.BlockSpec((BME, QKV), lambda i: (i, 0))],
        out_shape=[jax.ShapeDtypeStruct((E_PAD, DL), _f32),
                   jax.ShapeDtypeStruct((E_PAD, QKV), _f32)],
    )(ef_pad, edge_down_W.T, bc(edge_down_b, DL), Wqkvt, bqkv)

    sc_gather, sc_scatter = _sc_kernels()
    idx2 = jnp.concatenate([src_p, tgt_p])
    g2 = sc_gather(qkvn, idx2)

    oe, sst, bmax = pl.pallas_call(
        _attn_body,
        grid=(GE,),
        in_specs=[pl.BlockSpec((BME, QKV), lambda i: (i, 0)),
                  pl.BlockSpec((BME, QKV), lambda i: (i, 0)),
                  pl.BlockSpec((BME, QKV),
                               lambda i: (i + E_PAD // BME, 0))],
        out_specs=[pl.BlockSpec((BME, DL), lambda i: (i, 0)),
                   pl.BlockSpec((BME, 2 * H), lambda i: (i, 0)),
                   pl.BlockSpec((1, 1, 2 * H), lambda i: (i, 0, 0))],
        out_shape=[jax.ShapeDtypeStruct((E_PAD, DL), _f32),
                   jax.ShapeDtypeStruct((E_PAD, 2 * H), _f32),
                   jax.ShapeDtypeStruct((GE, 1, 2 * H), _f32)],
    )(qkve, g2, g2)

    mm = jnp.max(bmax, axis=(0, 1))
    m4 = jnp.maximum(mm[:H], mm[H:])
    m8 = bc(jnp.concatenate([m4, m4]), 2 * H)

    rows4 = pl.pallas_call(
        _expand_body,
        grid=(GE,),
        in_specs=[pl.BlockSpec((BME, 2 * H), lambda i: (i, 0)),
                  pl.BlockSpec((BME, DL), lambda i: (i, 2)),
                  full((8, 2 * H))],
        out_specs=pl.BlockSpec((4, BME, DL), lambda i: (0, i, 0)),
        out_shape=jax.ShapeDtypeStruct((4, E_PAD, DL), _f32),
    )(sst, qkve, m8)

    numden = sc_scatter(rows4.reshape(4 * E_PAD, DL), idx2,
                        jnp.zeros((N_PAD, DL), _f32))

    nn_pad = pl.pallas_call(
        _node_fin_body,
        grid=(GN,),
        in_specs=[pl.BlockSpec((BMN, DL), lambda i: (i, 0)),
                  pl.BlockSpec((BMN, DL), lambda i: (i + N_PAD // BMN, 0)),
                  pl.BlockSpec((BMN, DL), lambda i: (i, 0)),
                  pl.BlockSpec((BMN, DN), lambda i: (i, 0)),
                  full((DL, DL)), full((8, DL)),
                  full((8, DL)), full((8, DL)),
                  full((DL, DN)), full((8, DN)),
                  full((8, DN)), full((8, DN))],
        out_specs=pl.BlockSpec((BMN, DN), lambda i: (i, 0)),
        out_shape=jax.ShapeDtypeStruct((N_PAD, DN), _f32),
    )(numden, numden, nl, nf_pad, o_W.T, bc(o_b, DL),
      bc(sn_g, DL), bc(sn_b, DL), nupWt, bc(nupb, DN),
      bc(nn_g, DN), bc(nn_b, DN))

    ne_pad = pl.pallas_call(
        _edge_fin_body,
        grid=(GE,),
        in_specs=[pl.BlockSpec((BME, DL), lambda i: (i, 0)),
                  pl.BlockSpec((BME, DL), lambda i: (i, 0)),
                  pl.BlockSpec((BME, DE), lambda i: (i, 0)),
                  full((DL, DL)), full((8, DL)),
                  full((8, DL)), full((8, DL)),
                  full((DL, DE)), full((8, DE)),
                  full((8, DE)), full((8, DE))],
        out_specs=pl.BlockSpec((BME, DE), lambda i: (i, 0)),
        out_shape=jax.ShapeDtypeStruct((E_PAD, DE), _f32),
    )(oe, el, ef_pad, o_W.T, bc(o_b, DL),
      bc(sn_g, DL), bc(sn_b, DL), eupWt, bc(eupb, DE),
      bc(en_g, DE), bc(en_b, DE))

    return (nn_pad[:N], ne_pad[:E])


# speedup vs baseline: 154.6949x; 1.0441x over previous
import functools
import math

import jax
import jax.numpy as jnp
from jax import lax
from jax.experimental import pallas as pl
from jax.experimental.pallas import tpu as pltpu
from jax.experimental.pallas import tpu_sc as plsc

N = 10000
E = 160000
DN = 256
DE = 16
DL = 128
H = 4
DH = 32
QKV = 3 * DL

E_PAD = 163840
BME = 2048
GE = E_PAD // BME
N_PAD = 10240
BMN = 2048
GN = N_PAD // BMN

_NC = 2
_NS = 16
_NW = _NC * _NS
_CH = 128
_PER_W = E_PAD // _NW
_NCH = _PER_W // _CH
_NPS = N_PAD // _NS

_INV = 1.0 / math.sqrt(DH)
_f32 = jnp.float32


def _hsel():
    return (lax.broadcasted_iota(jnp.int32, (DL, H), 0) // DH ==
            lax.broadcasted_iota(jnp.int32, (DL, H), 1)).astype(_f32)


def _hselT():
    return (lax.broadcasted_iota(jnp.int32, (H, DL), 0) ==
            lax.broadcasted_iota(jnp.int32, (H, DL), 1) // DH).astype(_f32)


def _dsel():
    return (lax.broadcasted_iota(jnp.int32, (H, DL), 0) ==
            lax.broadcasted_iota(jnp.int32, (H, DL), 1)).astype(_f32)


def _densel():
    return (lax.broadcasted_iota(jnp.int32, (DL, DL), 0) ==
            lax.broadcasted_iota(jnp.int32, (DL, DL), 1) // DH).astype(_f32)


def _proj_body(x_ref, w1_ref, b1_ref, wq_ref, bq_ref, lat_ref, qkv_ref):
    lat = jnp.dot(x_ref[...], w1_ref[...],
                  preferred_element_type=_f32) + b1_ref[0:1, :]
    lat_ref[...] = lat
    qkv_ref[...] = jnp.dot(lat, wq_ref[...],
                           preferred_element_type=_f32) + bq_ref[0:1, :]


def _attn_body(qkv_ref, gs_ref, gt_ref, oe_ref, sst_ref, bm_ref):
    qkv = qkv_ref[...]
    qe = qkv[:, :DL]
    ke = qkv[:, DL:2 * DL]
    gs = gs_ref[...]
    gt = gt_ref[...]
    hsel = _hsel()
    hselT = _hselT()

    def hd(a, b):
        return jnp.dot(a * b, hsel, preferred_element_type=_f32) * _INV

    s1 = hd(qe, gs[:, DL:2 * DL])
    s2 = hd(qe, gt[:, DL:2 * DL])
    m = jnp.maximum(jnp.maximum(s1, s2), -1e9)
    a1 = jnp.exp(s1 - m)
    a2 = jnp.exp(s2 - m)
    r = 1.0 / (a1 + a2 + 1e-10)
    w1 = jnp.dot(a1 * r, hselT, preferred_element_type=_f32)
    w2 = jnp.dot(a2 * r, hselT, preferred_element_type=_f32)
    oe_ref[...] = w1 * gs[:, 2 * DL:] + w2 * gt[:, 2 * DL:]
    ss = hd(gs[:, :DL], ke)
    st = hd(gt[:, :DL], ke)
    sst = jnp.concatenate([ss, st], axis=1)
    sst_ref[...] = sst
    bm_ref[...] = jnp.max(sst, axis=0, keepdims=True).reshape(1, 1, 2 * H)


def _expand_body(sst_ref, v_ref, m_ref, out_ref):
    i = pl.program_id(0)
    sst = sst_ref[...]
    p = jnp.exp(sst - m_ref[0:1, :])
    rows = i * BME + lax.broadcasted_iota(jnp.int32, (BME, 2 * H), 0)
    p = jnp.where(rows < E, p, 0.0)
    ps = p[:, :H]
    pt = p[:, H:]
    hselT = _hselT()
    dsel = _dsel()
    v = v_ref[...]
    out_ref[0] = jnp.dot(ps, hselT, preferred_element_type=_f32) * v
    out_ref[1] = jnp.dot(pt, hselT, preferred_element_type=_f32) * v
    out_ref[2] = jnp.dot(ps, dsel, preferred_element_type=_f32)
    out_ref[3] = jnp.dot(pt, dsel, preferred_element_type=_f32)


def _ln(y, g, b):
    mu = jnp.mean(y, axis=1, keepdims=True)
    var = jnp.mean((y - mu) ** 2, axis=1, keepdims=True)
    return (y - mu) / jnp.sqrt(var + 1e-5) * g + b


def _node_fin_body(num_ref, den_ref, lat_ref, nf_ref, owt_ref, ob_ref,
                   sng_ref, snb_ref, upw_ref, upb_ref, nng_ref, nnb_ref,
                   out_ref):
    num = num_ref[...]
    den = den_ref[...]
    denf = jnp.dot(den, _densel(), preferred_element_type=_f32)
    on = num / (denf + 1e-20)
    x = jnp.dot(on, owt_ref[...], preferred_element_type=_f32) + ob_ref[0:1, :]
    z = _ln(lat_ref[...] + x, sng_ref[0:1, :], snb_ref[0:1, :])
    nu = jnp.dot(z, upw_ref[...], preferred_element_type=_f32) + upb_ref[0:1, :]
    out_ref[...] = _ln(nf_ref[...] + nu, nng_ref[0:1, :], nnb_ref[0:1, :])


def _edge_fin_body(oe_ref, lat_ref, ef_ref, owt_ref, ob_ref,
                   sng_ref, snb_ref, upw_ref, upb_ref, eng_ref, enb_ref,
                   out_ref):
    x = jnp.dot(oe_ref[...], owt_ref[...],
                preferred_element_type=_f32) + ob_ref[0:1, :]
    z = _ln(lat_ref[...] + x, sng_ref[0:1, :], snb_ref[0:1, :])
    eu = jnp.dot(z, upw_ref[...], preferred_element_type=_f32) + upb_ref[0:1, :]
    out_ref[...] = _ln(ef_ref[...] + eu, eng_ref[0:1, :], enb_ref[0:1, :])


def _sc_gather_body(tab, idx2, g2, i0, i1, r0b, r1b, gs0, gs1, ws0, ws1):
    cid = lax.axis_index("c")
    sid = lax.axis_index("s")
    wid = sid * _NC + cid
    base = wid * (2 * _PER_W)
    nch = (2 * _PER_W) // _CH
    ibufs = (i0, i1)
    rbufs = (r0b, r1b)
    gsems = (gs0, gs1)
    wsems = (ws0, ws1)

    pltpu.sync_copy(idx2.at[pl.ds(base, _CH)], i0)
    pltpu.async_copy(tab.at[i0], r0b, gs0)

    def outer(jj, c):
        for b in range(2):
            j = jj * 2 + b
            nb = 1 - b
            @pl.when(j + 1 < nch)
            def _():
                pltpu.sync_copy(idx2.at[pl.ds(base + (j + 1) * _CH, _CH)],
                                ibufs[nb])
            @pl.when(j >= 1)
            def _():
                pltpu.make_async_copy(rbufs[nb],
                                      g2.at[pl.ds(base, _CH)],
                                      wsems[nb]).wait()
            @pl.when(j + 1 < nch)
            def _():
                pltpu.async_copy(tab.at[ibufs[nb]], rbufs[nb], gsems[nb])
            pltpu.make_async_copy(tab.at[ibufs[b]], rbufs[b],
                                  gsems[b]).wait()
            pltpu.async_copy(rbufs[b], g2.at[pl.ds(base + j * _CH, _CH)],
                             wsems[b])
        return c

    lax.fori_loop(0, nch // 2, outer, 0)
    pltpu.make_async_copy(r1b, g2.at[pl.ds(base, _CH)], ws1).wait()


def _sc_scatter_body(rows4, idx2, z128, out, i0, i1, r0b, r1b, tab, ss0, ss1):
    cid = lax.axis_index("c")
    sid = lax.axis_index("s")
    r0 = sid * _NPS
    def zstep(j, c):
        rr = r0 + j * _CH
        pltpu.sync_copy(z128.at[pl.ds(rr, _CH)], r0b)
        pltpu.sync_copy(r0b, tab.at[pl.ds(rr, _CH)])
        return c
    lax.fori_loop(0, _NPS // _CH, zstep, 0)
    plsc.subcore_barrier()

    per_sub = (2 * E_PAD) // _NS
    nch = per_sub // _CH
    base_r = 2 * cid * E_PAD + sid * per_sub
    base_i = sid * per_sub
    ibufs = (i0, i1)
    rbufs = (r0b, r1b)
    ssems = (ss0, ss1)

    pltpu.sync_copy(idx2.at[pl.ds(base_i, _CH)], i0)
    pltpu.sync_copy(rows4.at[pl.ds(base_r, _CH)], r0b)

    def outer(jj, c):
        for b in range(2):
            j = jj * 2 + b
            nb = 1 - b
            pltpu.async_copy(rbufs[b], tab.at[ibufs[b]], ssems[b], add=True)
            @pl.when(j >= 1)
            def _():
                pltpu.make_async_copy(rbufs[nb], tab.at[ibufs[nb]],
                                      ssems[nb]).wait()
            @pl.when(j + 1 < nch)
            def _():
                pltpu.sync_copy(idx2.at[pl.ds(base_i + (j + 1) * _CH, _CH)],
                                ibufs[nb])
                pltpu.sync_copy(rows4.at[pl.ds(base_r + (j + 1) * _CH, _CH)],
                                rbufs[nb])
        return c

    lax.fori_loop(0, nch // 2, outer, 0)
    pltpu.make_async_copy(r1b, tab.at[i1], ss1).wait()
    plsc.subcore_barrier()
    def wstep(j, c):
        rr = r0 + j * _CH
        pltpu.sync_copy(tab.at[pl.ds(rr, _CH)], r0b)
        pltpu.sync_copy(r0b, out.at[pl.ds(cid * N_PAD + rr, _CH)])
        return c
    lax.fori_loop(0, _NPS // _CH, wstep, 0)


@functools.lru_cache(maxsize=None)
def _sc_kernels():
    mesh = plsc.VectorSubcoreMesh(core_axis_name="c", subcore_axis_name="s",
                                  num_cores=_NC, num_subcores=_NS)
    gather = pl.kernel(
        _sc_gather_body,
        out_type=jax.ShapeDtypeStruct((2 * E_PAD, QKV), _f32),
        mesh=mesh,
        scratch_types=[
            pltpu.VMEM((_CH,), jnp.int32),
            pltpu.VMEM((_CH,), jnp.int32),
            pltpu.VMEM((_CH, QKV), _f32),
            pltpu.VMEM((_CH, QKV), _f32),
            pltpu.SemaphoreType.DMA,
            pltpu.SemaphoreType.DMA,
            pltpu.SemaphoreType.DMA,
            pltpu.SemaphoreType.DMA,
        ],
    )
    scatter = pl.kernel(
        _sc_scatter_body,
        out_type=jax.ShapeDtypeStruct((2 * N_PAD, DL), _f32),
        mesh=mesh,
        scratch_types=[
            pltpu.VMEM((_CH,), jnp.int32),
            pltpu.VMEM((_CH,), jnp.int32),
            pltpu.VMEM((_CH, DL), _f32),
            pltpu.VMEM((_CH, DL), _f32),
            pltpu.VMEM_SHARED((N_PAD, DL), _f32),
            pltpu.SemaphoreType.DMA,
            pltpu.SemaphoreType.DMA,
        ],
    )
    return gather, scatter


def kernel(node_features, edge_features, node_down_W, node_down_b,
           edge_down_W, edge_down_b, q_W, q_b, k_W, k_b, v_W, v_b,
           o_W, o_b, sn_g, sn_b, node_up_W, node_up_b, edge_up_W, edge_up_b,
           node_gate, edge_gate, nn_g, nn_b, en_g, en_b, edge_index):
    def bc(v, w):
        return jnp.broadcast_to(v.reshape(1, w), (8, w))

    Wqkvt = jnp.concatenate([q_W.T, k_W.T, v_W.T], axis=1)
    bqkv = bc(jnp.concatenate([q_b, k_b, v_b]), QKV)
    g_n = jax.nn.sigmoid(node_gate)[0]
    g_e = jax.nn.sigmoid(edge_gate)[0]
    nupWt = node_up_W.T * g_n
    nupb = node_up_b * g_n
    eupWt = edge_up_W.T * g_e
    eupb = edge_up_b * g_e
    ef_pad = jnp.concatenate(
        [edge_features, jnp.zeros((E_PAD - E, DE), _f32)], axis=0)
    nf_pad = jnp.concatenate(
        [node_features, jnp.zeros((N_PAD - N, DN), _f32)], axis=0)
    src = edge_index[0]
    tgt = edge_index[1]
    zpad = jnp.zeros((E_PAD - E,), jnp.int32)
    src_p = jnp.concatenate([src, zpad])
    tgt_p = jnp.concatenate([tgt, zpad])

    full = lambda shape: pl.BlockSpec(shape, lambda i: (0, 0))

    nl, qkvn = pl.pallas_call(
        _proj_body,
        grid=(GN,),
        in_specs=[pl.BlockSpec((BMN, DN), lambda i: (i, 0)),
                  full((DN, DL)), full((8, DL)),
                  full((DL, QKV)), full((8, QKV))],
        out_specs=[pl.BlockSpec((BMN, DL), lambda i: (i, 0)),
                   pl.BlockSpec((BMN, QKV), lambda i: (i, 0))],
        out_shape=[jax.ShapeDtypeStruct((N_PAD, DL), _f32),
                   jax.ShapeDtypeStruct((N_PAD, QKV), _f32)],
    )(nf_pad, node_down_W.T, bc(node_down_b, DL), Wqkvt, bqkv)

    el, qkve = pl.pallas_call(
        _proj_body,
        grid=(GE,),
        in_specs=[pl.BlockSpec((BME, DE), lambda i: (i, 0)),
                  full((DE, DL)), full((8, DL)),
                  full((DL, QKV)), full((8, QKV))],
        out_specs=[pl.BlockSpec((BME, DL), lambda i: (i, 0)),
                   pl.BlockSpec((BME, QKV), lambda i: (i, 0))],
        out_shape=[jax.ShapeDtypeStruct((E_PAD, DL), _f32),
                   jax.ShapeDtypeStruct((E_PAD, QKV), _f32)],
    )(ef_pad, edge_down_W.T, bc(edge_down_b, DL), Wqkvt, bqkv)

    sc_gather, sc_scatter = _sc_kernels()
    idx2 = jnp.concatenate([src_p, tgt_p])
    g2 = sc_gather(qkvn, idx2)

    oe, sst, bmax = pl.pallas_call(
        _attn_body,
        grid=(GE,),
        in_specs=[pl.BlockSpec((BME, QKV), lambda i: (i, 0)),
                  pl.BlockSpec((BME, QKV), lambda i: (i, 0)),
                  pl.BlockSpec((BME, QKV),
                               lambda i: (i + E_PAD // BME, 0))],
        out_specs=[pl.BlockSpec((BME, DL), lambda i: (i, 0)),
                   pl.BlockSpec((BME, 2 * H), lambda i: (i, 0)),
                   pl.BlockSpec((1, 1, 2 * H), lambda i: (i, 0, 0))],
        out_shape=[jax.ShapeDtypeStruct((E_PAD, DL), _f32),
                   jax.ShapeDtypeStruct((E_PAD, 2 * H), _f32),
                   jax.ShapeDtypeStruct((GE, 1, 2 * H), _f32)],
    )(qkve, g2, g2)

    mm = jnp.max(bmax, axis=(0, 1))
    m4 = jnp.maximum(mm[:H], mm[H:])
    m8 = bc(jnp.concatenate([m4, m4]), 2 * H)

    rows4 = pl.pallas_call(
        _expand_body,
        grid=(GE,),
        in_specs=[pl.BlockSpec((BME, 2 * H), lambda i: (i, 0)),
                  pl.BlockSpec((BME, DL), lambda i: (i, 2)),
                  full((8, 2 * H))],
        out_specs=pl.BlockSpec((4, BME, DL), lambda i: (0, i, 0)),
        out_shape=jax.ShapeDtypeStruct((4, E_PAD, DL), _f32),
    )(sst, qkve, m8)

    numden = sc_scatter(rows4.reshape(4 * E_PAD, DL), idx2,
                        jnp.zeros((N_PAD, DL), _f32))

    nn_pad = pl.pallas_call(
        _node_fin_body,
        grid=(GN,),
        in_specs=[pl.BlockSpec((BMN, DL), lambda i: (i, 0)),
                  pl.BlockSpec((BMN, DL), lambda i: (i + N_PAD // BMN, 0)),
                  pl.BlockSpec((BMN, DL), lambda i: (i, 0)),
                  pl.BlockSpec((BMN, DN), lambda i: (i, 0)),
                  full((DL, DL)), full((8, DL)),
                  full((8, DL)), full((8, DL)),
                  full((DL, DN)), full((8, DN)),
                  full((8, DN)), full((8, DN))],
        out_specs=pl.BlockSpec((BMN, DN), lambda i: (i, 0)),
        out_shape=jax.ShapeDtypeStruct((N_PAD, DN), _f32),
    )(numden, numden, nl, nf_pad, o_W.T, bc(o_b, DL),
      bc(sn_g, DL), bc(sn_b, DL), nupWt, bc(nupb, DN),
      bc(nn_g, DN), bc(nn_b, DN))

    ne_pad = pl.pallas_call(
        _edge_fin_body,
        grid=(GE,),
        in_specs=[pl.BlockSpec((BME, DL), lambda i: (i, 0)),
                  pl.BlockSpec((BME, DL), lambda i: (i, 0)),
                  pl.BlockSpec((BME, DE), lambda i: (i, 0)),
                  full((DL, DL)), full((8, DL)),
                  full((8, DL)), full((8, DL)),
                  full((DL, DE)), full((8, DE)),
                  full((8, DE)), full((8, DE))],
        out_specs=pl.BlockSpec((BME, DE), lambda i: (i, 0)),
        out_shape=jax.ShapeDtypeStruct((E_PAD, DE), _f32),
    )(oe, el, ef_pad, o_W.T, bc(o_b, DL),
      bc(sn_g, DL), bc(sn_b, DL), eupWt, bc(eupb, DE),
      bc(en_g, DE), bc(en_b, DE))

    return (nn_pad[:N], ne_pad[:E])
